# Initial kernel scaffold; baseline (speedup 1.0000x reference)
import jax
import jax.numpy as jnp
from jax.experimental import pallas as pl


def kernel(feats, dW1, db1, dlng, dlnb, dW2, db2, dbng, dbnb, cW1, cb1, clng, clnb, cW2, cb2, cbng, cbnb, convW1, convb1, convW2, convb2, linW, linb, edge_index, idx):
    raise NotImplementedError("write your pallas kernel here")


# speedup vs baseline: 6.7563x; 6.7563x over previous
import functools

import jax
import jax.numpy as jnp
from jax import lax
from jax.experimental import pallas as pl
from jax.experimental.pallas import tpu as pltpu
from jax.experimental.pallas import tpu_sc as plsc

N_DRUG = 8000
TAIL = 2000
N = 10000
E = 160000
HID = 512
OUT1 = 256
OUT_CH = 128

F32 = jnp.float32
NC = 2
NS = 16
CH = 128
ROWS_PER_TILE = 640


def _proj_drug_body(x_ref, w1_ref, b1_ref, lg_ref, lb_ref, w2_ref, b2_ref,
                    h2_ref, cs_ref, cq_ref):
    i = pl.program_id(0)
    h = lax.dot_general(x_ref[...], w1_ref[...], (((1,), (1,)), ((), ())),
                        preferred_element_type=F32) + b1_ref[...]
    mu = jnp.mean(h, axis=-1, keepdims=True)
    v = jnp.mean((h - mu) ** 2, axis=-1, keepdims=True)
    h = (h - mu) / jnp.sqrt(v + 1e-5) * lg_ref[...] + lb_ref[...]
    h = jnp.maximum(h, 0.0)
    h2 = lax.dot_general(h, w2_ref[...], (((1,), (1,)), ((), ())),
                         preferred_element_type=F32) + b2_ref[...]
    h2_ref[...] = h2

    @pl.when(i == 0)
    def _():
        cs_ref[...] = jnp.zeros_like(cs_ref)
        cq_ref[...] = jnp.zeros_like(cq_ref)

    cs_ref[...] += jnp.sum(h2, axis=0, keepdims=True)
    cq_ref[...] += jnp.sum(h2 * h2, axis=0, keepdims=True)


def _proj_drug(feats, w1, b1, lg, lb, w2, b2):
    bm = 1000
    grid = (N_DRUG // bm,)
    return pl.pallas_call(
        _proj_drug_body,
        grid=grid,
        in_specs=[
            pl.BlockSpec((bm, TAIL), lambda i: (i, 0)),
            pl.BlockSpec((HID, TAIL), lambda i: (0, 0)),
            pl.BlockSpec((1, HID), lambda i: (0, 0)),
            pl.BlockSpec((1, HID), lambda i: (0, 0)),
            pl.BlockSpec((1, HID), lambda i: (0, 0)),
            pl.BlockSpec((HID, HID), lambda i: (0, 0)),
            pl.BlockSpec((1, HID), lambda i: (0, 0)),
        ],
        out_specs=[
            pl.BlockSpec((bm, HID), lambda i: (i, 0)),
            pl.BlockSpec((1, HID), lambda i: (0, 0)),
            pl.BlockSpec((1, HID), lambda i: (0, 0)),
        ],
        out_shape=[
            jax.ShapeDtypeStruct((N_DRUG, HID), F32),
            jax.ShapeDtypeStruct((1, HID), F32),
            jax.ShapeDtypeStruct((1, HID), F32),
        ],
    )(feats, w1, b1, lg, lb, w2, b2)


def _proj_chem_body(x_ref, w1_ref, b1_ref, lg_ref, lb_ref, w2_ref, b2_ref,
                    h2_ref, cs_ref, cq_ref, acc_ref):
    i = pl.program_id(0)
    nsteps = pl.num_programs(0)

    @pl.when(i == 0)
    def _():
        acc_ref[...] = jnp.zeros_like(acc_ref)

    acc_ref[...] += lax.dot_general(
        x_ref[...], w1_ref[...], (((0,), (0,)), ((), ())),
        preferred_element_type=F32)

    @pl.when(i == nsteps - 1)
    def _():
        h = acc_ref[...] + b1_ref[...]
        mu = jnp.mean(h, axis=-1, keepdims=True)
        v = jnp.mean((h - mu) ** 2, axis=-1, keepdims=True)
        h = (h - mu) / jnp.sqrt(v + 1e-5) * lg_ref[...] + lb_ref[...]
        h = jnp.maximum(h, 0.0)
        h2 = lax.dot_general(h, w2_ref[...], (((1,), (1,)), ((), ())),
                             preferred_element_type=F32) + b2_ref[...]
        h2_ref[...] = h2
        cs_ref[...] = jnp.sum(h2, axis=0, keepdims=True)
        cq_ref[...] = jnp.sum(h2 * h2, axis=0, keepdims=True)


def _proj_chem(feats, w1t, b1, lg, lb, w2, b2):
    bk = 1000
    grid = (N_DRUG // bk,)
    return pl.pallas_call(
        _proj_chem_body,
        grid=grid,
        in_specs=[
            pl.BlockSpec((bk, TAIL), lambda i: (i, 0)),
            pl.BlockSpec((bk, HID), lambda i: (i, 0)),
            pl.BlockSpec((1, HID), lambda i: (0, 0)),
            pl.BlockSpec((1, HID), lambda i: (0, 0)),
            pl.BlockSpec((1, HID), lambda i: (0, 0)),
            pl.BlockSpec((HID, HID), lambda i: (0, 0)),
            pl.BlockSpec((1, HID), lambda i: (0, 0)),
        ],
        out_specs=[
            pl.BlockSpec((TAIL, HID), lambda i: (0, 0)),
            pl.BlockSpec((1, HID), lambda i: (0, 0)),
            pl.BlockSpec((1, HID), lambda i: (0, 0)),
        ],
        out_shape=[
            jax.ShapeDtypeStruct((TAIL, HID), F32),
            jax.ShapeDtypeStruct((1, HID), F32),
            jax.ShapeDtypeStruct((1, HID), F32),
        ],
        scratch_shapes=[pltpu.VMEM((TAIL, HID), F32)],
    )(feats, w1t, b1, lg, lb, w2, b2)


def _bn_gcn1_body(h2_ref, cs_ref, cq_ref, g_ref, b_ref, p0_ref, p1_ref,
                  w_ref, ya_ref, yb_ref):
    i = pl.program_id(0)
    dom = i >= 8
    nrows = jnp.where(dom, float(TAIL), float(N_DRUG))
    cs = cs_ref[...]
    cq = cq_ref[...]
    mu = jnp.where(dom, cs[1:2, :], cs[0:1, :]) / nrows
    var = jnp.where(dom, cq[1:2, :], cq[0:1, :]) / nrows - mu * mu
    g = jnp.where(dom, g_ref[1:2, :], g_ref[0:1, :])
    b = jnp.where(dom, b_ref[1:2, :], b_ref[0:1, :])
    x = (h2_ref[...] - mu) / jnp.sqrt(var + 1e-5) * g + b
    x = jnp.maximum(x, 0.0)
    xw = lax.dot_general(x, w_ref[...], (((1,), (1,)), ((), ())),
                         preferred_element_type=F32)
    dis = lax.rsqrt(p0_ref[...] + p1_ref[...] + 1.0)
    y = xw * dis
    ya_ref[...] = y[:, :OUT_CH]
    yb_ref[...] = y[:, OUT_CH:]


def _bn_gcn1(h2, cs2, cq2, g2, b2, p0, p1, w):
    bm = 1000
    grid = (N // bm,)
    return pl.pallas_call(
        _bn_gcn1_body,
        grid=grid,
        in_specs=[
            pl.BlockSpec((bm, HID), lambda i: (i, 0)),
            pl.BlockSpec((2, HID), lambda i: (0, 0)),
            pl.BlockSpec((2, HID), lambda i: (0, 0)),
            pl.BlockSpec((2, HID), lambda i: (0, 0)),
            pl.BlockSpec((2, HID), lambda i: (0, 0)),
            pl.BlockSpec((bm, 1), lambda i: (i, 0)),
            pl.BlockSpec((bm, 1), lambda i: (i, 0)),
            pl.BlockSpec((OUT1, HID), lambda i: (0, 0)),
        ],
        out_specs=[
            pl.BlockSpec((bm, OUT_CH), lambda i: (i, 0)),
            pl.BlockSpec((bm, OUT_CH), lambda i: (i, 0)),
        ],
        out_shape=[
            jax.ShapeDtypeStruct((N, OUT_CH), F32),
            jax.ShapeDtypeStruct((N, OUT_CH), F32),
        ],
    )(h2, cs2, cq2, g2, b2, p0, p1, w)


def _gcn2_in_body(aa_ref, ab_ref, ya_ref, yb_ref, p0_ref, p1_ref, b1_ref,
                  w_ref, o_ref):
    dis = lax.rsqrt(p0_ref[...] + p1_ref[...] + 1.0)
    s = jnp.concatenate([aa_ref[...] + ya_ref[...],
                         ab_ref[...] + yb_ref[...]], axis=1)
    x1 = jnp.maximum(dis * s + b1_ref[...], 0.0)
    xw = lax.dot_general(x1, w_ref[...], (((1,), (1,)), ((), ())),
                         preferred_element_type=F32)
    o_ref[...] = xw * dis


def _gcn2_in(aggA, aggB, yA, yB, p0, p1, b1, w):
    bm = 1000
    grid = (N // bm,)
    return pl.pallas_call(
        _gcn2_in_body,
        grid=grid,
        in_specs=[
            pl.BlockSpec((bm, OUT_CH), lambda i: (i, 0)),
            pl.BlockSpec((bm, OUT_CH), lambda i: (i, 0)),
            pl.BlockSpec((bm, OUT_CH), lambda i: (i, 0)),
            pl.BlockSpec((bm, OUT_CH), lambda i: (i, 0)),
            pl.BlockSpec((bm, 1), lambda i: (i, 0)),
            pl.BlockSpec((bm, 1), lambda i: (i, 0)),
            pl.BlockSpec((1, OUT1), lambda i: (0, 0)),
            pl.BlockSpec((OUT_CH, OUT1), lambda i: (0, 0)),
        ],
        out_specs=pl.BlockSpec((bm, OUT_CH), lambda i: (i, 0)),
        out_shape=jax.ShapeDtypeStruct((N, OUT_CH), F32),
    )(aggA, aggB, yA, yB, p0, p1, b1, w)


def _final_body(a0_ref, a1_ref, y2_ref, p0_ref, p1_ref, b2_ref,
                lw_ref, lb_ref, df_ref, z_ref):
    dis = lax.rsqrt(p0_ref[...] + p1_ref[...] + 1.0)
    s = a0_ref[...] + a1_ref[...] + y2_ref[...]
    df = dis * s + b2_ref[...]
    df_ref[...] = df
    z_ref[...] = lax.dot_general(df, lw_ref[...], (((1,), (1,)), ((), ())),
                                 preferred_element_type=F32) + lb_ref[...]


def _final(agg2P0, agg2P1, y2, p0, p1, b2, lw, lb):
    bm = 1000
    grid = (N // bm,)
    return pl.pallas_call(
        _final_body,
        grid=grid,
        in_specs=[
            pl.BlockSpec((bm, OUT_CH), lambda i: (i, 0)),
            pl.BlockSpec((bm, OUT_CH), lambda i: (i, 0)),
            pl.BlockSpec((bm, OUT_CH), lambda i: (i, 0)),
            pl.BlockSpec((bm, 1), lambda i: (i, 0)),
            pl.BlockSpec((bm, 1), lambda i: (i, 0)),
            pl.BlockSpec((1, OUT_CH), lambda i: (0, 0)),
            pl.BlockSpec((OUT_CH, OUT_CH), lambda i: (0, 0)),
            pl.BlockSpec((1, OUT_CH), lambda i: (0, 0)),
        ],
        out_specs=[
            pl.BlockSpec((bm, OUT_CH), lambda i: (i, 0)),
            pl.BlockSpec((bm, OUT_CH), lambda i: (i, 0)),
        ],
        out_shape=[
            jax.ShapeDtypeStruct((N, OUT_CH), F32),
            jax.ShapeDtypeStruct((N, OUT_CH), F32),
        ],
    )(agg2P0, agg2P1, y2, p0, p1, b2, lw, lb)


_MESH = plsc.VectorSubcoreMesh(core_axis_name="c", subcore_axis_name="s")


N_PAD = 10240


def _deg_body(dst_hbm, zer_hbm, one_hbm, out0, out1, acc, idxv, onev, sem):
    c = lax.axis_index("c")
    s = lax.axis_index("s")
    sl = pl.ds(640 * s, 640)
    pltpu.sync_copy(zer_hbm, acc.at[sl])
    pltpu.sync_copy(one_hbm, onev)
    plsc.subcore_barrier()

    nk = 39 + jnp.where(s == 0, 1, 0)

    @pl.loop(0, nk)
    def _(k):
        base = pl.multiple_of(c * (E // 2) + (k * NS + s) * CH, CH)
        pltpu.sync_copy(dst_hbm.at[pl.ds(base, CH)], idxv)
        pltpu.sync_copy(onev, acc.at[idxv], add=True)

    plsc.subcore_barrier()

    @pl.when(c == 0)
    def _():
        pltpu.sync_copy(acc.at[sl], out0.at[sl])

    @pl.when(c == 1)
    def _():
        pltpu.sync_copy(acc.at[sl], out1.at[sl])


def _deg_hist(dst, zeros640, ones128):
    fn = pl.kernel(
        _deg_body,
        out_type=(jax.ShapeDtypeStruct((N_PAD,), F32),
                  jax.ShapeDtypeStruct((N_PAD,), F32)),
        mesh=_MESH,
        scratch_types=[
            pltpu.VMEM_SHARED((N_PAD,), F32),
            pltpu.VMEM((CH,), jnp.int32),
            pltpu.VMEM((CH,), F32),
            pltpu.SemaphoreType.DMA,
        ],
    )
    return fn(dst, zeros640, ones128)


def _make_seg_body(width):
    def body(ya_hbm, yb_hbm, src_hbm, dst_hbm, zer_hbm, outA, outB,
             acc, idxs, idxd, rows, sem):
        c = lax.axis_index("c")
        s = lax.axis_index("s")
        pltpu.sync_copy(zer_hbm, acc.at[pl.ds(s * ROWS_PER_TILE, ROWS_PER_TILE)])
        plsc.subcore_barrier()

        nk = (E // CH) // NS + jnp.where(s < (E // CH) % NS, 1, 0)

        @pl.loop(0, nk)
        def _(k):
            base = pl.multiple_of((k * NS + s) * CH, CH)
            pltpu.sync_copy(src_hbm.at[pl.ds(base, CH)], idxs)
            pltpu.sync_copy(dst_hbm.at[pl.ds(base, CH)], idxd)

            @pl.when(c == 0)
            def _():
                pltpu.async_copy(ya_hbm.at[idxs], rows, sem).wait()

            @pl.when(c == 1)
            def _():
                pltpu.async_copy(yb_hbm.at[idxs], rows, sem).wait()

            pltpu.sync_copy(rows, acc.at[idxd], add=True)

        plsc.subcore_barrier()
        lo = pl.ds(s * ROWS_PER_TILE, ROWS_PER_TILE)

        @pl.when(c == 0)
        def _():
            pltpu.sync_copy(acc.at[lo], outA.at[lo])

        @pl.when(c == 1)
        def _():
            pltpu.sync_copy(acc.at[lo], outB.at[lo])

    return body


def _seg_sum(width, yA, yB, src, dst, zeros_rows):
    fn = pl.kernel(
        _make_seg_body(width),
        out_type=(jax.ShapeDtypeStruct((N_PAD, width), F32),
                  jax.ShapeDtypeStruct((N_PAD, width), F32)),
        mesh=_MESH,
        scratch_types=[
            pltpu.VMEM_SHARED((N_PAD, width), F32),
            pltpu.VMEM((CH,), jnp.int32),
            pltpu.VMEM((CH,), jnp.int32),
            pltpu.VMEM((CH, width), F32),
            pltpu.SemaphoreType.DMA,
        ],
    )
    return fn(yA, yB, src, dst, zeros_rows)


def _seg_partial_body(y_hbm, src_hbm, dst_hbm, zer_hbm, out0, out1,
                      acc, idxs, idxd, rows, sem):
    c = lax.axis_index("c")
    s = lax.axis_index("s")
    sl = pl.ds(s * ROWS_PER_TILE, ROWS_PER_TILE)
    pltpu.sync_copy(zer_hbm, acc.at[sl])
    plsc.subcore_barrier()

    half_chunks = (E // 2) // CH
    nk = half_chunks // NS + jnp.where(s < half_chunks % NS, 1, 0)

    @pl.loop(0, nk)
    def _(k):
        base = pl.multiple_of(c * (E // 2) + (k * NS + s) * CH, CH)
        pltpu.sync_copy(src_hbm.at[pl.ds(base, CH)], idxs)
        pltpu.sync_copy(dst_hbm.at[pl.ds(base, CH)], idxd)
        pltpu.async_copy(y_hbm.at[idxs], rows, sem).wait()
        pltpu.sync_copy(rows, acc.at[idxd], add=True)

    plsc.subcore_barrier()

    @pl.when(c == 0)
    def _():
        pltpu.sync_copy(acc.at[sl], out0.at[sl])

    @pl.when(c == 1)
    def _():
        pltpu.sync_copy(acc.at[sl], out1.at[sl])


def _seg_partial(y, src, dst, zeros_rows):
    fn = pl.kernel(
        _seg_partial_body,
        out_type=(jax.ShapeDtypeStruct((N_PAD, OUT_CH), F32),
                  jax.ShapeDtypeStruct((N_PAD, OUT_CH), F32)),
        mesh=_MESH,
        scratch_types=[
            pltpu.VMEM_SHARED((N_PAD, OUT_CH), F32),
            pltpu.VMEM((CH,), jnp.int32),
            pltpu.VMEM((CH,), jnp.int32),
            pltpu.VMEM((CH, OUT_CH), F32),
            pltpu.SemaphoreType.DMA,
        ],
    )
    return fn(y, src, dst, zeros_rows)


def _edge_body(z_hbm, src_hbm, dst_hbm, outL, outR,
               idxs, idxd, bufL, bufR, semL, semR):
    c = lax.axis_index("c")
    s = lax.axis_index("s")
    w = s * NC + c
    nw = NC * NS
    nk = (E // CH) // nw + jnp.where(w < (E // CH) % nw, 1, 0)

    @pl.loop(0, nk)
    def _(k):
        base = pl.multiple_of((k * nw + w) * CH, CH)
        pltpu.sync_copy(src_hbm.at[pl.ds(base, CH)], idxs)
        pltpu.sync_copy(dst_hbm.at[pl.ds(base, CH)], idxd)
        dl = pltpu.async_copy(z_hbm.at[idxs], bufL, semL)
        dr = pltpu.async_copy(z_hbm.at[idxd], bufR, semR)
        dl.wait()
        dr.wait()
        pltpu.sync_copy(bufL, outL.at[pl.ds(base, CH)])
        pltpu.sync_copy(bufR, outR.at[pl.ds(base, CH)])


def _edge_gather(z, src, dst):
    fn = pl.kernel(
        _edge_body,
        out_type=(jax.ShapeDtypeStruct((E, OUT_CH), F32),
                  jax.ShapeDtypeStruct((E, OUT_CH), F32)),
        mesh=_MESH,
        scratch_types=[
            pltpu.VMEM((CH,), jnp.int32),
            pltpu.VMEM((CH,), jnp.int32),
            pltpu.VMEM((CH, OUT_CH), F32),
            pltpu.VMEM((CH, OUT_CH), F32),
            pltpu.SemaphoreType.DMA,
            pltpu.SemaphoreType.DMA,
        ],
    )
    return fn(z, src, dst)


def kernel(feats, dW1, db1, dlng, dlnb, dW2, db2, dbng, dbnb, cW1, cb1, clng,
           clnb, cW2, cb2, cbng, cbnb, convW1, convb1, convW2, convb2, linW,
           linb, edge_index, idx):
    src = edge_index[0]
    dst = edge_index[1]
    r = lambda v: v.reshape(1, -1)

    zeros640 = jnp.zeros((640,), F32)
    ones128 = jnp.ones((CH,), F32)
    d0, d1 = _deg_hist(dst, zeros640, ones128)
    p0 = d0[:N].reshape(N, 1)
    p1 = d1[:N].reshape(N, 1)

    h2_d, cs_d, cq_d = _proj_drug(feats, dW1, r(db1), r(dlng), r(dlnb),
                                  dW2, r(db2))
    h2_c, cs_c, cq_c = _proj_chem(feats, cW1.T, r(cb1), r(clng), r(clnb),
                                  cW2, r(cb2))

    h2 = jnp.concatenate([h2_d, h2_c], axis=0)
    cs2 = jnp.concatenate([cs_d, cs_c], axis=0)
    cq2 = jnp.concatenate([cq_d, cq_c], axis=0)
    g2 = jnp.stack([dbng, cbng], axis=0)
    b2 = jnp.stack([dbnb, cbnb], axis=0)

    yA, yB = _bn_gcn1(h2, cs2, cq2, g2, b2, p0, p1, convW1)

    zrows128 = jnp.zeros((ROWS_PER_TILE, OUT_CH), F32)
    aggA, aggB = _seg_sum(OUT_CH, yA, yB, src, dst, zrows128)
    aggA, aggB = aggA[:N], aggB[:N]

    y2 = _gcn2_in(aggA, aggB, yA, yB, p0, p1, r(convb1), convW2)

    agg2P0, agg2P1 = _seg_partial(y2, src, dst, zrows128)
    agg2P0, agg2P1 = agg2P0[:N], agg2P1[:N]

    drug_f, z = _final(agg2P0, agg2P1, y2, p0, p1, r(convb2),
                       linW, r(linb))

    eL, eR = _edge_gather(z, src, dst)
    edge_feat = jnp.concatenate([eL, eR], axis=1)
    return (drug_f, edge_feat, idx)
